# baseline (speedup 1.0000x reference)
import functools

import jax
import jax.numpy as jnp
from jax import lax
from jax.experimental import pallas as pl
from jax.experimental.pallas import tpu as pltpu
from jax.experimental.pallas import tpu_sc as plsc

BATCH, N, K, D = 8, 2048, 256, 512
NC, NS = 2, 16
BATCH_PER_CORE = BATCH // NC
TILES_PER_BATCH = NS // BATCH_PER_CORE
SCAT_ROWS = K // TILES_PER_BATCH
ZERO_ROWS = BATCH_PER_CORE * N // NS
ZBUF_ROWS = 64
ZERO_REPS = ZERO_ROWS // ZBUF_ROWS


def _sc_body(x_hbm, idx_hbm, out_hbm, zeros_v, rows_v, idx_v, sem):
    c = lax.axis_index("c")
    s = lax.axis_index("s")
    b = c * BATCH_PER_CORE + s // TILES_PER_BATCH
    chunk = b * K + (s % TILES_PER_BATCH) * SCAT_ROWS
    zrow0 = (c * NS + s) * ZERO_ROWS

    zv = jnp.zeros((16,), jnp.float32)
    def fill_row(r, carry):
        for cj in range(D // 16):
            zeros_v[r, 16 * cj:16 * (cj + 1)] = zv
        return carry
    lax.fori_loop(0, ZBUF_ROWS, fill_row, 0)

    zero_dmas = [
        pltpu.async_copy(
            zeros_v, out_hbm.at[pl.ds(zrow0 + j * ZBUF_ROWS, ZBUF_ROWS)], sem)
        for j in range(ZERO_REPS)
    ]

    pltpu.sync_copy(idx_hbm.at[pl.ds(chunk, SCAT_ROWS)], idx_v)
    pltpu.sync_copy(x_hbm.at[pl.ds(chunk, SCAT_ROWS)], rows_v)
    off = (b * N).astype(jnp.int32)
    for j in range(SCAT_ROWS // 16):
        idx_v[16 * j:16 * (j + 1)] = idx_v[16 * j:16 * (j + 1)] + off

    for dma in zero_dmas:
        dma.wait()
    plsc.subcore_barrier()

    pltpu.async_copy(rows_v, out_hbm.at[idx_v], sem).wait()


_sc_scatter = functools.partial(
    pl.kernel,
    mesh=plsc.VectorSubcoreMesh(core_axis_name="c", subcore_axis_name="s"),
    out_type=jax.ShapeDtypeStruct((BATCH * N, D), jnp.float32),
    scratch_types=[
        pltpu.VMEM((ZBUF_ROWS, D), jnp.float32),
        pltpu.VMEM((SCAT_ROWS, D), jnp.float32),
        pltpu.VMEM((SCAT_ROWS,), jnp.int32),
        pltpu.SemaphoreType.DMA,
    ],
)(_sc_body)


def kernel(A, X, idx_batch):
    x_flat = X.reshape(BATCH * K, D)
    idx_flat = idx_batch.reshape(BATCH * K).astype(jnp.int32)
    out = _sc_scatter(x_flat, idx_flat)
    return (A, out.reshape(BATCH, N, D))
